# baseline (speedup 1.0000x reference)
import functools

import jax
import jax.numpy as jnp
from jax import lax
from jax.experimental import pallas as pl
from jax.experimental.pallas import tpu as pltpu
from jax.experimental.pallas import tpu_sc as plsc

C = 1024
H = 128
HW = H // 2
SPW = 16
LANES = 16

def _sc_topology():
    info = plsc.get_sparse_core_info()
    return info.num_cores, info.num_subcores


def _r_table_body(p_ref, w1_ref, b1_ref, w2_ref, b2_ref, rp_ref):
    p = p_ref[...]
    h = p + jnp.maximum(
        jnp.dot(p, w1_ref[...], preferred_element_type=jnp.float32) + b1_ref[...],
        0.0,
    )
    r = h + jnp.maximum(
        jnp.dot(h, w2_ref[...], preferred_element_type=jnp.float32) + b2_ref[...],
        0.0,
    )
    lo = lax.bitcast_convert_type(r[:, :HW].astype(jnp.bfloat16), jnp.uint16)
    hi = lax.bitcast_convert_type(r[:, HW:].astype(jnp.bfloat16), jnp.uint16)
    packed = lo.astype(jnp.uint32) | (hi.astype(jnp.uint32) << 16)
    rp_ref[...] = lax.bitcast_convert_type(packed, jnp.int32)


def _compute_r_packed(pret, w1, b1, w2, b2):
    return pl.pallas_call(
        _r_table_body,
        out_shape=jax.ShapeDtypeStruct((C, HW), jnp.int32),
    )(pret, w1, b1.reshape(1, H), w2, b2.reshape(1, H))


def _log_vec(sv):
    bits = plsc.bitcast(sv, jnp.int32)
    ex = jnp.right_shift(bits, 23) - 127
    mant = plsc.bitcast(
        jnp.bitwise_or(jnp.bitwise_and(bits, 0x7FFFFF), 0x3F800000), jnp.float32
    )
    big = mant > 1.4142135
    mant = jnp.where(big, mant * 0.5, mant)
    ex = ex + jnp.where(big, 1, 0)
    t = mant - 1.0
    p = 1.0 / 8.0 - t * (1.0 / 9.0)
    p = 1.0 / 7.0 - t * p
    p = 1.0 / 6.0 - t * p
    p = 1.0 / 5.0 - t * p
    p = 1.0 / 4.0 - t * p
    p = 1.0 / 3.0 - t * p
    p = 1.0 / 2.0 - t * p
    p = t * (1.0 - t * p)
    return ex.astype(jnp.float32) * 0.69314718 + p


def _tree_sum(vs):
    while len(vs) > 1:
        vs = [vs[i] + vs[i + 1] for i in range(0, len(vs) - 1, 2)] + (
            [vs[-1]] if len(vs) % 2 else []
        )
    return vs[0]


def _tree_max(vs):
    while len(vs) > 1:
        vs = [jnp.maximum(vs[i], vs[i + 1]) for i in range(0, len(vs) - 1, 2)] + (
            [vs[-1]] if len(vs) % 2 else []
        )
    return vs[0]


def _make_sc_kernel(n_tokens):
    NC, NS = _sc_topology()
    NW = NC * NS
    tpw = n_tokens // NW
    blk = 16
    nblk = tpw // blk
    orows = n_tokens * SPW // H
    orpw = orows // NW
    mesh = plsc.VectorSubcoreMesh(core_axis_name="c", subcore_axis_name="s")

    @functools.partial(
        pl.kernel,
        mesh=mesh,
        out_type=jax.ShapeDtypeStruct((orows, H), jnp.float32),
        scratch_types=[
            pltpu.VMEM((tpw,), jnp.int32),
            pltpu.VMEM((tpw,), jnp.int32),
            pltpu.VMEM((tpw, H), jnp.int32),
            pltpu.VMEM((tpw, H), jnp.float32),
            pltpu.VMEM((C * HW,), jnp.int32),
            pltpu.VMEM((orpw, H), jnp.float32),
            pltpu.SemaphoreType.DMA,
            pltpu.SemaphoreType.DMA,
            pltpu.SemaphoreType.DMA,
        ],
        compiler_params=pltpu.CompilerParams(needs_layout_passes=False),
    )
    def sc_kernel(text_hbm, w2s_hbm, rp_hbm, emb_hbm, out_hbm,
                  idx_v, idx8_v, stg_v, obs_v, rp_v, res_v,
                  sem_a, sem_b, sem_c):
        wid = lax.axis_index("s") * NC + lax.axis_index("c")
        base = wid * tpw
        pltpu.sync_copy(text_hbm.at[pl.ds(base, tpw)], idx_v)
        for c in range(tpw // LANES):
            idx8_v[pl.ds(c * LANES, LANES)] = jnp.right_shift(
                idx_v[pl.ds(c * LANES, LANES)], 3
            )
        cp_r = pltpu.async_copy(rp_hbm, rp_v, sem_a)
        cp_w = pltpu.async_copy(w2s_hbm.at[idx8_v], stg_v, sem_b)
        cp_e = pltpu.async_copy(emb_hbm.at[idx_v], obs_v, sem_c)
        cp_r.wait()
        cp_w.wait()
        cp_e.wait()

        lanes = lax.iota(jnp.int32, LANES)

        def block_body(b, carry):
            trows = b * blk + lanes
            wvec = idx_v[pl.ds(b * blk, blk)]
            offv = (wvec & 7) * SPW
            sidx = [
                plsc.load_gather(stg_v, [trows, offv + k]) * HW
                for k in range(SPW)
            ]
            accs = [jnp.zeros((LANES,), jnp.float32) for _ in range(SPW)]
            for w in range(HW):
                ow = plsc.load_gather(
                    obs_v, [trows, jnp.full((LANES,), w, jnp.int32)]
                )
                ow64 = plsc.load_gather(
                    obs_v, [trows, jnp.full((LANES,), w + HW, jnp.int32)]
                )
                for k in range(SPW):
                    word = plsc.load_gather(rp_v, [sidx[k] + w])
                    flo = plsc.bitcast(jnp.left_shift(word, 16), jnp.float32)
                    fhi = plsc.bitcast(
                        jnp.bitwise_and(word, jnp.int32(-65536)), jnp.float32
                    )
                    accs[k] = accs[k] + flo * ow + fhi * ow64

            m = _tree_max(accs)
            xs = [a - m for a in accs]
            ls = _log_vec(_tree_sum([jnp.exp(x) for x in xs]))
            rowv = jnp.right_shift(trows, 3)
            colbase = (trows & 7) * SPW
            for k in range(SPW):
                plsc.store_scatter(res_v, [rowv, colbase + k], xs[k] - ls)
            return carry

        lax.fori_loop(0, nblk, block_body, 0)
        pltpu.sync_copy(res_v, out_hbm.at[pl.ds(wid * orpw, orpw)])

    return sc_kernel


def kernel(text, word2state, preterminal_emb, terminal_emb, W1, b1, W2, b2):
    b, t = text.shape
    n = b * t
    rp = _compute_r_packed(preterminal_emb, W1, b1, W2, b2).reshape(C * HW)
    w2s_r = word2state.reshape(word2state.shape[0] * SPW // H, H)
    flat_text = text.reshape(n).astype(jnp.int32)
    out = _make_sc_kernel(n)(flat_text, w2s_r, rp, terminal_emb)
    return out.reshape(b, t, SPW)


# speedup vs baseline: 1.4287x; 1.4287x over previous
import functools

import jax
import jax.numpy as jnp
from jax import lax
from jax.experimental import pallas as pl
from jax.experimental.pallas import tpu as pltpu
from jax.experimental.pallas import tpu_sc as plsc

C = 1024
H = 128
HW = H // 2
RSTR = HW + 1
OSTR = H + 1
SPW = 16
LANES = 16

def _sc_topology():
    info = plsc.get_sparse_core_info()
    return info.num_cores, info.num_subcores


def _r_table_body(p_ref, w1_ref, b1_ref, w2_ref, b2_ref, rp_ref):
    p = p_ref[...]
    h = p + jnp.maximum(
        jnp.dot(p, w1_ref[...], preferred_element_type=jnp.float32) + b1_ref[...],
        0.0,
    )
    r = h + jnp.maximum(
        jnp.dot(h, w2_ref[...], preferred_element_type=jnp.float32) + b2_ref[...],
        0.0,
    )
    lo = lax.bitcast_convert_type(r[:, :HW].astype(jnp.bfloat16), jnp.uint16)
    hi = lax.bitcast_convert_type(r[:, HW:].astype(jnp.bfloat16), jnp.uint16)
    packed = lo.astype(jnp.uint32) | (hi.astype(jnp.uint32) << 16)
    rp_ref[...] = lax.bitcast_convert_type(packed, jnp.int32)


def _compute_r_packed(pret, w1, b1, w2, b2):
    return pl.pallas_call(
        _r_table_body,
        out_shape=jax.ShapeDtypeStruct((C, HW), jnp.int32),
    )(pret, w1, b1.reshape(1, H), w2, b2.reshape(1, H))


def _log_vec(sv):
    bits = plsc.bitcast(sv, jnp.int32)
    ex = jnp.right_shift(bits, 23) - 127
    mant = plsc.bitcast(
        jnp.bitwise_or(jnp.bitwise_and(bits, 0x7FFFFF), 0x3F800000), jnp.float32
    )
    big = mant > 1.4142135
    mant = jnp.where(big, mant * 0.5, mant)
    ex = ex + jnp.where(big, 1, 0)
    t = mant - 1.0
    p = 1.0 / 8.0 - t * (1.0 / 9.0)
    p = 1.0 / 7.0 - t * p
    p = 1.0 / 6.0 - t * p
    p = 1.0 / 5.0 - t * p
    p = 1.0 / 4.0 - t * p
    p = 1.0 / 3.0 - t * p
    p = 1.0 / 2.0 - t * p
    p = t * (1.0 - t * p)
    return ex.astype(jnp.float32) * 0.69314718 + p


def _tree_sum(vs):
    while len(vs) > 1:
        vs = [vs[i] + vs[i + 1] for i in range(0, len(vs) - 1, 2)] + (
            [vs[-1]] if len(vs) % 2 else []
        )
    return vs[0]


def _tree_max(vs):
    while len(vs) > 1:
        vs = [jnp.maximum(vs[i], vs[i + 1]) for i in range(0, len(vs) - 1, 2)] + (
            [vs[-1]] if len(vs) % 2 else []
        )
    return vs[0]


def _make_sc_kernel(n_tokens):
    NC, NS = _sc_topology()
    NW = NC * NS
    tpw = n_tokens // NW
    blk = 16
    nblk = tpw // blk
    orows = n_tokens * SPW // H
    orpw = orows // NW
    mesh = plsc.VectorSubcoreMesh(core_axis_name="c", subcore_axis_name="s")

    @functools.partial(
        pl.kernel,
        mesh=mesh,
        out_type=jax.ShapeDtypeStruct((orows, H), jnp.float32),
        scratch_types=[
            pltpu.VMEM((tpw,), jnp.int32),
            pltpu.VMEM((tpw,), jnp.int32),
            pltpu.VMEM((tpw, H), jnp.int32),
            pltpu.VMEM((tpw, H), jnp.float32),
            pltpu.VMEM((tpw * OSTR,), jnp.float32),
            pltpu.VMEM((C * RSTR,), jnp.int32),
            pltpu.VMEM((orpw, H), jnp.float32),
            pltpu.SemaphoreType.DMA,
            pltpu.SemaphoreType.DMA,
            pltpu.SemaphoreType.DMA,
        ],
        compiler_params=pltpu.CompilerParams(needs_layout_passes=False),
    )
    def sc_kernel(text_hbm, w2s_hbm, rp_hbm, emb_hbm, out_hbm,
                  idx_v, idx8_v, stg_v, obs_v, obsp_v, rp_v, res_v,
                  sem_a, sem_b, sem_c):
        wid = lax.axis_index("s") * NC + lax.axis_index("c")
        base = wid * tpw
        pltpu.sync_copy(text_hbm.at[pl.ds(base, tpw)], idx_v)
        for c in range(tpw // LANES):
            idx8_v[pl.ds(c * LANES, LANES)] = jnp.right_shift(
                idx_v[pl.ds(c * LANES, LANES)], 3
            )
        cp_r = pltpu.async_copy(rp_hbm, rp_v, sem_a)
        cp_w = pltpu.async_copy(w2s_hbm.at[idx8_v], stg_v, sem_b)
        cp_e = pltpu.async_copy(emb_hbm.at[idx_v], obs_v, sem_c)
        cp_r.wait()
        cp_w.wait()
        cp_e.wait()

        def relay_body(t, carry):
            for c in range(H // LANES):
                obsp_v[pl.ds(t * OSTR + c * LANES, LANES)] = obs_v[
                    t, pl.ds(c * LANES, LANES)
                ]
            return carry

        lax.fori_loop(0, tpw, relay_body, 0)

        lanes = lax.iota(jnp.int32, LANES)

        def block_body(b, carry):
            trows = b * blk + lanes
            wvec = idx_v[pl.ds(b * blk, blk)]
            offv = (wvec & 7) * SPW
            sidx = [
                plsc.load_gather(stg_v, [trows, offv + k]) * RSTR
                for k in range(SPW)
            ]
            tr_o = trows * OSTR
            accs = [jnp.zeros((LANES,), jnp.float32) for _ in range(SPW)]
            for w in range(HW):
                ow = plsc.load_gather(obsp_v, [tr_o + w])
                ow64 = plsc.load_gather(obsp_v, [tr_o + (w + HW)])
                for k in range(SPW):
                    word = plsc.load_gather(rp_v, [sidx[k] + w])
                    flo = plsc.bitcast(jnp.left_shift(word, 16), jnp.float32)
                    fhi = plsc.bitcast(
                        jnp.bitwise_and(word, jnp.int32(-65536)), jnp.float32
                    )
                    accs[k] = accs[k] + flo * ow + fhi * ow64

            m = _tree_max(accs)
            xs = [a - m for a in accs]
            ls = _log_vec(_tree_sum([jnp.exp(x) for x in xs]))
            rowv = jnp.right_shift(trows, 3)
            colbase = (trows & 7) * SPW
            for k in range(SPW):
                plsc.store_scatter(res_v, [rowv, colbase + k], xs[k] - ls)
            return carry

        lax.fori_loop(0, nblk, block_body, 0)
        pltpu.sync_copy(res_v, out_hbm.at[pl.ds(wid * orpw, orpw)])

    return sc_kernel


def kernel(text, word2state, preterminal_emb, terminal_emb, W1, b1, W2, b2):
    b, t = text.shape
    n = b * t
    rp2d = _compute_r_packed(preterminal_emb, W1, b1, W2, b2)
    rp = jnp.pad(rp2d, ((0, 0), (0, RSTR - HW))).reshape(C * RSTR)
    w2s_r = word2state.reshape(word2state.shape[0] * SPW // H, H)
    flat_text = text.reshape(n).astype(jnp.int32)
    out = _make_sc_kernel(n)(flat_text, w2s_r, rp, terminal_emb)
    return out.reshape(b, t, SPW)


# speedup vs baseline: 1.4359x; 1.0050x over previous
import functools

import jax
import jax.numpy as jnp
from jax import lax
from jax.experimental import pallas as pl
from jax.experimental.pallas import tpu as pltpu
from jax.experimental.pallas import tpu_sc as plsc

C = 1024
H = 128
HW = H // 2
RSTR = HW + 1
OSTR = H + 1
SPW = 16
LANES = 16

def _sc_topology():
    info = plsc.get_sparse_core_info()
    return info.num_cores, info.num_subcores


def _r_table_body(p_ref, w1_ref, b1_ref, w2_ref, b2_ref, rp_ref):
    p = p_ref[...]
    h = p + jnp.maximum(
        jnp.dot(p, w1_ref[...], preferred_element_type=jnp.float32) + b1_ref[...],
        0.0,
    )
    r = h + jnp.maximum(
        jnp.dot(h, w2_ref[...], preferred_element_type=jnp.float32) + b2_ref[...],
        0.0,
    )
    lo = lax.bitcast_convert_type(r[:, :HW].astype(jnp.bfloat16), jnp.uint16)
    hi = lax.bitcast_convert_type(r[:, HW:].astype(jnp.bfloat16), jnp.uint16)
    packed = lo.astype(jnp.uint32) | (hi.astype(jnp.uint32) << 16)
    rp_ref[...] = lax.bitcast_convert_type(packed, jnp.int32)


def _compute_r_packed(pret, w1, b1, w2, b2):
    return pl.pallas_call(
        _r_table_body,
        out_shape=jax.ShapeDtypeStruct((C, HW), jnp.int32),
    )(pret, w1, b1.reshape(1, H), w2, b2.reshape(1, H))


def _log_vec(sv):
    bits = plsc.bitcast(sv, jnp.int32)
    ex = jnp.right_shift(bits, 23) - 127
    mant = plsc.bitcast(
        jnp.bitwise_or(jnp.bitwise_and(bits, 0x7FFFFF), 0x3F800000), jnp.float32
    )
    big = mant > 1.4142135
    mant = jnp.where(big, mant * 0.5, mant)
    ex = ex + jnp.where(big, 1, 0)
    t = mant - 1.0
    p = 1.0 / 8.0 - t * (1.0 / 9.0)
    p = 1.0 / 7.0 - t * p
    p = 1.0 / 6.0 - t * p
    p = 1.0 / 5.0 - t * p
    p = 1.0 / 4.0 - t * p
    p = 1.0 / 3.0 - t * p
    p = 1.0 / 2.0 - t * p
    p = t * (1.0 - t * p)
    return ex.astype(jnp.float32) * 0.69314718 + p


def _tree_sum(vs):
    while len(vs) > 1:
        vs = [vs[i] + vs[i + 1] for i in range(0, len(vs) - 1, 2)] + (
            [vs[-1]] if len(vs) % 2 else []
        )
    return vs[0]


def _tree_max(vs):
    while len(vs) > 1:
        vs = [jnp.maximum(vs[i], vs[i + 1]) for i in range(0, len(vs) - 1, 2)] + (
            [vs[-1]] if len(vs) % 2 else []
        )
    return vs[0]


def _make_sc_kernel(n_tokens):
    NC, NS = _sc_topology()
    NW = NC * NS
    tpw = n_tokens // NW
    blk = 16
    nblk = tpw // blk
    orows = n_tokens * SPW // H
    orpw = orows // NW
    mesh = plsc.VectorSubcoreMesh(core_axis_name="c", subcore_axis_name="s")

    @functools.partial(
        pl.kernel,
        mesh=mesh,
        out_type=jax.ShapeDtypeStruct((orows, H), jnp.float32),
        scratch_types=[
            pltpu.VMEM((tpw,), jnp.int32),
            pltpu.VMEM((tpw,), jnp.int32),
            pltpu.VMEM((tpw, H), jnp.int32),
            pltpu.VMEM((tpw, H), jnp.float32),
            pltpu.VMEM((tpw * OSTR,), jnp.float32),
            pltpu.VMEM((C * RSTR,), jnp.int32),
            pltpu.VMEM((orpw, H), jnp.float32),
            pltpu.SemaphoreType.DMA,
            pltpu.SemaphoreType.DMA,
            pltpu.SemaphoreType.DMA,
        ],
        compiler_params=pltpu.CompilerParams(needs_layout_passes=False),
    )
    def sc_kernel(text_hbm, w2s_hbm, rp_hbm, emb_hbm, out_hbm,
                  idx_v, idx8_v, stg_v, obs_v, obsp_v, rp_v, res_v,
                  sem_a, sem_b, sem_c):
        wid = lax.axis_index("s") * NC + lax.axis_index("c")
        base = wid * tpw
        pltpu.sync_copy(text_hbm.at[pl.ds(base, tpw)], idx_v)
        for c in range(tpw // LANES):
            idx8_v[pl.ds(c * LANES, LANES)] = jnp.right_shift(
                idx_v[pl.ds(c * LANES, LANES)], 3
            )
        cp_r = pltpu.async_copy(rp_hbm, rp_v, sem_a)
        cp_w = pltpu.async_copy(w2s_hbm.at[idx8_v], stg_v, sem_b)
        cp_e = pltpu.async_copy(emb_hbm.at[idx_v], obs_v, sem_c)
        cp_r.wait()
        cp_w.wait()
        cp_e.wait()

        def relay_body(t, carry):
            for c in range(H // LANES):
                obsp_v[pl.ds(t * OSTR + c * LANES, LANES)] = obs_v[
                    t, pl.ds(c * LANES, LANES)
                ]
            return carry

        lax.fori_loop(0, tpw, relay_body, 0)

        lanes = lax.iota(jnp.int32, LANES)

        @plsc.parallel_loop(0, nblk)
        def block_body(b):
            trows = b * blk + lanes
            wvec = idx_v[pl.ds(b * blk, blk)]
            offv = (wvec & 7) * SPW
            sidx = [
                plsc.load_gather(stg_v, [trows, offv + k]) * RSTR
                for k in range(SPW)
            ]
            tr_o = trows * OSTR
            accs = [jnp.zeros((LANES,), jnp.float32) for _ in range(SPW)]
            for w in range(HW):
                ow = plsc.load_gather(obsp_v, [tr_o + w])
                ow64 = plsc.load_gather(obsp_v, [tr_o + (w + HW)])
                for k in range(SPW):
                    word = plsc.load_gather(rp_v, [sidx[k] + w])
                    flo = plsc.bitcast(jnp.left_shift(word, 16), jnp.float32)
                    fhi = plsc.bitcast(
                        jnp.bitwise_and(word, jnp.int32(-65536)), jnp.float32
                    )
                    accs[k] = accs[k] + flo * ow + fhi * ow64

            m = _tree_max(accs)
            xs = [a - m for a in accs]
            ls = _log_vec(_tree_sum([jnp.exp(x) for x in xs]))
            rowv = jnp.right_shift(trows, 3)
            colbase = (trows & 7) * SPW
            for k in range(SPW):
                plsc.store_scatter(res_v, [rowv, colbase + k], xs[k] - ls)

        pltpu.sync_copy(res_v, out_hbm.at[pl.ds(wid * orpw, orpw)])

    return sc_kernel


def kernel(text, word2state, preterminal_emb, terminal_emb, W1, b1, W2, b2):
    b, t = text.shape
    n = b * t
    rp2d = _compute_r_packed(preterminal_emb, W1, b1, W2, b2)
    rp = jnp.pad(rp2d, ((0, 0), (0, RSTR - HW))).reshape(C * RSTR)
    w2s_r = word2state.reshape(word2state.shape[0] * SPW // H, H)
    flat_text = text.reshape(n).astype(jnp.int32)
    out = _make_sc_kernel(n)(flat_text, w2s_r, rp, terminal_emb)
    return out.reshape(b, t, SPW)


# speedup vs baseline: 1.5545x; 1.0827x over previous
import functools

import jax
import jax.numpy as jnp
from jax import lax
from jax.experimental import pallas as pl
from jax.experimental.pallas import tpu as pltpu
from jax.experimental.pallas import tpu_sc as plsc

C = 1024
H = 128
SPW = 16
LANES = 16
NCHUNK = H // LANES


def _sc_topology():
    info = plsc.get_sparse_core_info()
    return info.num_cores, info.num_subcores


def _r_table_body(p_ref, w1_ref, b1_ref, w2_ref, b2_ref, r_ref):
    p = p_ref[...]
    h = p + jnp.maximum(
        jnp.dot(p, w1_ref[...], preferred_element_type=jnp.float32) + b1_ref[...],
        0.0,
    )
    r_ref[...] = h + jnp.maximum(
        jnp.dot(h, w2_ref[...], preferred_element_type=jnp.float32) + b2_ref[...],
        0.0,
    )


def _compute_r_table(pret, w1, b1, w2, b2):
    return pl.pallas_call(
        _r_table_body,
        out_shape=jax.ShapeDtypeStruct((C, H), jnp.float32),
    )(pret, w1, b1.reshape(1, H), w2, b2.reshape(1, H))


def _log_vec(sv):
    bits = plsc.bitcast(sv, jnp.int32)
    ex = jnp.right_shift(bits, 23) - 127
    mant = plsc.bitcast(
        jnp.bitwise_or(jnp.bitwise_and(bits, 0x7FFFFF), 0x3F800000), jnp.float32
    )
    big = mant > 1.4142135
    mant = jnp.where(big, mant * 0.5, mant)
    ex = ex + jnp.where(big, 1, 0)
    t = mant - 1.0
    p = 1.0 / 8.0 - t * (1.0 / 9.0)
    p = 1.0 / 7.0 - t * p
    p = 1.0 / 6.0 - t * p
    p = 1.0 / 5.0 - t * p
    p = 1.0 / 4.0 - t * p
    p = 1.0 / 3.0 - t * p
    p = 1.0 / 2.0 - t * p
    p = t * (1.0 - t * p)
    return ex.astype(jnp.float32) * 0.69314718 + p


def _tree_sum(vs):
    while len(vs) > 1:
        vs = [vs[i] + vs[i + 1] for i in range(0, len(vs) - 1, 2)] + (
            [vs[-1]] if len(vs) % 2 else []
        )
    return vs[0]


def _tree_max(vs):
    while len(vs) > 1:
        vs = [jnp.maximum(vs[i], vs[i + 1]) for i in range(0, len(vs) - 1, 2)] + (
            [vs[-1]] if len(vs) % 2 else []
        )
    return vs[0]


def _make_sc_kernel(n_tokens):
    NC, NS = _sc_topology()
    NW = NC * NS
    tpw = n_tokens // NW
    blk = 16
    nblk = tpw // blk
    mesh = plsc.VectorSubcoreMesh(core_axis_name="c", subcore_axis_name="s")

    @functools.partial(
        pl.kernel,
        mesh=mesh,
        out_type=jax.ShapeDtypeStruct((n_tokens, SPW), jnp.float32),
        scratch_types=[
            pltpu.VMEM((tpw,), jnp.int32),
            pltpu.VMEM((tpw, SPW), jnp.int32),
            pltpu.VMEM((tpw * SPW,), jnp.int32),
            pltpu.VMEM((tpw, H), jnp.float32),
            pltpu.VMEM((blk * SPW, H), jnp.float32),
            pltpu.VMEM((blk * SPW, H), jnp.float32),
            pltpu.VMEM((SPW, LANES), jnp.float32),
            pltpu.VMEM((blk, SPW), jnp.float32),
            pltpu.VMEM((tpw, SPW), jnp.float32),
            pltpu.SemaphoreType.DMA,
            pltpu.SemaphoreType.DMA,
        ],
        compiler_params=pltpu.CompilerParams(
            needs_layout_passes=False, use_tc_tiling_on_sc=False
        ),
    )
    def sc_kernel(text_hbm, w2s_hbm, r_hbm, emb_hbm, out_hbm,
                  idx_v, st2_v, stf_v, obs_v, rr0_v, rr1_v, tr_v, lt_v, res_v,
                  sem0, sem1):
        wid = lax.axis_index("s") * NC + lax.axis_index("c")
        base = wid * tpw
        pltpu.sync_copy(text_hbm.at[pl.ds(base, tpw)], idx_v)
        pltpu.async_copy(w2s_hbm.at[idx_v], st2_v, sem0).wait()
        pltpu.async_copy(emb_hbm.at[idx_v], obs_v, sem0).wait()

        def flatten_body(t, carry):
            stf_v[pl.ds(t * SPW, SPW)] = st2_v[t]
            return carry

        lax.fori_loop(0, tpw, flatten_body, 0)

        lanes = lax.iota(jnp.int32, LANES)
        half = blk * SPW // 2

        def start_blk(b, rr, sem):
            off = b * blk * SPW
            pltpu.async_copy(
                r_hbm.at[stf_v.at[pl.ds(off, half)]], rr.at[pl.ds(0, half)], sem
            )
            pltpu.async_copy(
                r_hbm.at[stf_v.at[pl.ds(off + half, half)]],
                rr.at[pl.ds(half, half)], sem,
            )

        def wait_blk(b, rr, sem):
            off = b * blk * SPW
            pltpu.make_async_copy(
                r_hbm.at[stf_v.at[pl.ds(off, half)]], rr.at[pl.ds(0, half)], sem
            ).wait()
            pltpu.make_async_copy(
                r_hbm.at[stf_v.at[pl.ds(off + half, half)]],
                rr.at[pl.ds(half, half)], sem,
            ).wait()

        def compute_blk(b, rr):
            def token_body(j, carry):
                t = b * blk + j
                o = [obs_v[t, pl.ds(c * LANES, LANES)] for c in range(NCHUNK)]
                for k in range(SPW):
                    tr_v[k] = _tree_sum(
                        [o[c] * rr[j * SPW + k, pl.ds(c * LANES, LANES)]
                         for c in range(NCHUNK)]
                    )
                cols = [
                    plsc.load_gather(tr_v, [lanes, jnp.full((LANES,), c, jnp.int32)])
                    for c in range(LANES)
                ]
                lt_v[j] = _tree_sum(cols)
                return carry

            lax.fori_loop(0, blk, token_body, 0)

            vks = [
                plsc.load_gather(lt_v, [lanes, jnp.full((LANES,), k, jnp.int32)])
                for k in range(SPW)
            ]
            m = _tree_max(vks)
            xs = [vk - m for vk in vks]
            ls = _log_vec(_tree_sum([jnp.exp(x) for x in xs]))
            rows = b * blk + lanes
            for k in range(SPW):
                plsc.store_scatter(
                    res_v, [rows, jnp.full((LANES,), k, jnp.int32)], xs[k] - ls
                )

        start_blk(0, rr0_v, sem0)
        start_blk(1, rr1_v, sem1)

        def pair_body(p, carry):
            b0 = 2 * p
            b1 = 2 * p + 1
            wait_blk(b0, rr0_v, sem0)
            compute_blk(b0, rr0_v)

            @pl.when(b0 + 2 < nblk)
            def _():
                start_blk(b0 + 2, rr0_v, sem0)

            wait_blk(b1, rr1_v, sem1)
            compute_blk(b1, rr1_v)

            @pl.when(b1 + 2 < nblk)
            def _():
                start_blk(b1 + 2, rr1_v, sem1)

            return carry

        lax.fori_loop(0, nblk // 2, pair_body, 0)
        pltpu.sync_copy(res_v, out_hbm.at[pl.ds(base, tpw)])

    return sc_kernel


def kernel(text, word2state, preterminal_emb, terminal_emb, W1, b1, W2, b2):
    b, t = text.shape
    n = b * t
    r_table = _compute_r_table(preterminal_emb, W1, b1, W2, b2)
    flat_text = text.reshape(n).astype(jnp.int32)
    out = _make_sc_kernel(n)(flat_text, word2state, r_table, terminal_emb)
    return out.reshape(b, t, SPW)
